# baseline (speedup 1.0000x reference)
import jax
import jax.numpy as jnp
from jax.experimental import pallas as pl
from jax.experimental.pallas import tpu as pltpu

B, N, C, IN, HID, ATOMS = 2, 4096, 1024, 128, 64, 64
TILE_A = 1024
NTA = N // TILE_A
TILE_B = 1024
NTB = N // TILE_B


def _phase_a(imp_ref, x_ref, inc_ref, nmw_ref, nmb_ref, cmw_ref, cmb_ref,
             atoms_ref, qw_ref, qb_ref, kw_ref, kb_ref,
             h0_ref, pc_ref, ent_ref, acc_ref, nmw2_ref, cmw2_ref):
    i = pl.program_id(0)

    @pl.when(i == 0)
    def _():
        nmw2_ref[...] = jnp.zeros((B * HID, B * IN), jnp.float32)
        cmw2_ref[...] = jnp.zeros((B * HID, B * IN), jnp.float32)
        nmw2_ref[0:HID, 0:IN] = nmw_ref[...]
        nmw2_ref[HID:B * HID, IN:B * IN] = nmw_ref[...]
        cmw2_ref[0:HID, 0:IN] = cmw_ref[...]
        cmw2_ref[HID:B * HID, IN:B * IN] = cmw_ref[...]

    gate = jax.nn.sigmoid(imp_ref[0, :])
    x_cols = jnp.concatenate([x_ref[0], x_ref[1]], axis=1) * gate[:, None]

    nmb2 = jnp.concatenate([nmb_ref[0, :], nmb_ref[0, :]])
    h0_cols = jax.lax.dot_general(
        x_cols, nmw2_ref[...], (((1,), (1,)), ((), ())),
        preferred_element_type=jnp.float32) + nmb2
    h0_ref[...] = h0_cols

    xc_cols = jax.lax.dot_general(
        x_cols, cmw2_ref[...], (((1,), (1,)), ((), ())),
        preferred_element_type=jnp.float32)
    contrib = jax.lax.dot_general(
        inc_ref[...], xc_cols, (((0,), (0,)), ((), ())),
        preferred_element_type=jnp.float32)

    @pl.when(i == 0)
    def _():
        acc_ref[...] = contrib

    @pl.when(i > 0)
    def _():
        acc_ref[...] += contrib

    @pl.when(i == NTA - 1)
    def _():
        acc = acc_ref[...]
        h2 = jnp.concatenate([acc[:, :HID], acc[:, HID:]], axis=0) \
            + cmb_ref[0, :]
        q = jnp.dot(h2, qw_ref[...].T, preferred_element_type=jnp.float32) \
            + qb_ref[0, :]
        k = jnp.dot(atoms_ref[...], kw_ref[...].T,
                    preferred_element_type=jnp.float32) + kb_ref[0, :]
        attn = jax.lax.dot_general(
            q, k, (((1,), (1,)), ((), ())),
            preferred_element_type=jnp.float32) * (HID ** -0.5)
        m = jnp.max(attn, axis=-1, keepdims=True)
        e = jnp.exp(attn - m)
        s = jnp.sum(e, axis=-1, keepdims=True)
        w = e / s
        pc = jnp.dot(w, atoms_ref[...],
                     preferred_element_type=jnp.float32)
        pc_ref[...] = jnp.concatenate([pc[:C], pc[C:]], axis=1)
        ent = jnp.mean(-jnp.sum(w * jnp.log(w + 1e-6), axis=-1))
        ent_ref[...] = ent.reshape(1, 1)


def _phase_b(adj_ref, h0_ref, inc_ref, pc_ref, sw_ref, sb_ref,
             c1w_ref, c1b_ref, c2w_ref, c2b_ref, pcg_ref, pcb_ref,
             fw_ref, fb_ref, ng_ref, nb_ref, out_ref):
    agg_cols = jnp.dot(adj_ref[...], h0_ref[...],
                       preferred_element_type=jnp.float32)
    pn_cols = jnp.dot(inc_ref[...], pc_ref[...],
                      preferred_element_type=jnp.float32)
    agg = jnp.concatenate([agg_cols[:, :HID], agg_cols[:, HID:]], axis=0)
    pn = jnp.concatenate([pn_cols[:, :HID], pn_cols[:, HID:]], axis=0)
    sur = agg - pn

    err = jnp.sqrt(jnp.sum(sur * sur, axis=-1, keepdims=True))
    conf = 1.0 / (1.0 + err)
    ps = jnp.dot(sur, sw_ref[...].T, preferred_element_type=jnp.float32) \
        + sb_ref[0, :]
    t = jnp.maximum(
        jax.lax.dot_general(jnp.abs(sur), c1w_ref[...],
                            (((1,), (1,)), ((), ())),
                            preferred_element_type=jnp.float32)
        + c1b_ref[0, :], 0.0)
    lc = jax.nn.sigmoid(
        jnp.sum(t * c2w_ref[0:1, :], axis=-1, keepdims=True) + c2b_ref[0, 0])
    gated = ps * (conf * lc)

    h = gated + agg
    mu = jnp.mean(h, axis=-1, keepdims=True)
    va = jnp.mean((h - mu) ** 2, axis=-1, keepdims=True)
    processed = (h - mu) * jax.lax.rsqrt(va + 1e-5) * pcg_ref[0, :] \
        + pcb_ref[0, :]

    fw = fw_ref[...]
    comb = jax.lax.dot_general(processed, fw[:, :HID],
                               (((1,), (1,)), ((), ())),
                               preferred_element_type=jnp.float32) \
        + jax.lax.dot_general(pn, fw[:, HID:],
                              (((1,), (1,)), ((), ())),
                              preferred_element_type=jnp.float32) \
        + fb_ref[0, :]
    mu2 = jnp.mean(comb, axis=-1, keepdims=True)
    va2 = jnp.mean((comb - mu2) ** 2, axis=-1, keepdims=True)
    out = (comb - mu2) * jax.lax.rsqrt(va2 + 1e-5) * ng_ref[0, :] \
        + nb_ref[0, :]
    out_ref[0] = out[0:TILE_B].T
    out_ref[1] = out[TILE_B:B * TILE_B].T


def _full(shape):
    return pl.BlockSpec(shape, lambda i: tuple(0 for _ in shape))


def kernel(x_nodes, adjacency, incidence, node_importance,
           nm_w, nm_b, cm_w, cm_b, atoms, q_w, q_b, k_w, k_b,
           s_w, s_b, c1_w, c1_b, c2_w, c2_b, pc_g, pc_b, f_w, f_b, n_g, n_b):
    imp2 = node_importance.reshape(1, N)
    r = lambda v: v.reshape(1, -1)

    h0_nm, pc_nm, ent = pl.pallas_call(
        _phase_a,
        grid=(NTA,),
        in_specs=[
            pl.BlockSpec((1, TILE_A), lambda i: (0, i)),
            pl.BlockSpec((B, TILE_A, IN), lambda i: (0, i, 0)),
            pl.BlockSpec((TILE_A, C), lambda i: (i, 0)),
            _full((HID, IN)), _full((1, HID)),
            _full((HID, IN)), _full((1, HID)),
            _full((ATOMS, HID)),
            _full((HID, HID)), _full((1, HID)),
            _full((HID, HID)), _full((1, HID)),
        ],
        out_specs=[
            pl.BlockSpec((TILE_A, B * HID), lambda i: (i, 0)),
            _full((C, B * HID)),
            _full((1, 1)),
        ],
        out_shape=[
            jax.ShapeDtypeStruct((N, B * HID), jnp.float32),
            jax.ShapeDtypeStruct((C, B * HID), jnp.float32),
            jax.ShapeDtypeStruct((1, 1), jnp.float32),
        ],
        scratch_shapes=[pltpu.VMEM((C, B * HID), jnp.float32),
                        pltpu.VMEM((B * HID, B * IN), jnp.float32),
                        pltpu.VMEM((B * HID, B * IN), jnp.float32)],
    )(imp2, x_nodes, incidence, nm_w, r(nm_b), cm_w, r(cm_b),
      atoms, q_w, r(q_b), k_w, r(k_b))

    out = pl.pallas_call(
        _phase_b,
        grid=(NTB,),
        in_specs=[
            pl.BlockSpec((TILE_B, N), lambda i: (i, 0)),
            _full((N, B * HID)),
            pl.BlockSpec((TILE_B, C), lambda i: (i, 0)),
            _full((C, B * HID)),
            _full((HID, HID)), _full((1, HID)),
            _full((HID // 4, HID)), _full((1, HID // 4)),
            _full((1, HID // 4)), _full((1, 1)),
            _full((1, HID)), _full((1, HID)),
            _full((HID, B * HID)), _full((1, HID)),
            _full((1, HID)), _full((1, HID)),
        ],
        out_specs=pl.BlockSpec((B, HID, TILE_B), lambda i: (0, 0, i)),
        out_shape=jax.ShapeDtypeStruct((B, HID, N), jnp.float32),
    )(adjacency, h0_nm, incidence, pc_nm, s_w, r(s_b),
      c1_w, r(c1_b), c2_w, c2_b.reshape(1, 1), r(pc_g), r(pc_b),
      f_w, r(f_b), r(n_g), r(n_b))

    return jnp.transpose(out, (0, 2, 1)), ent.reshape(())


# speedup vs baseline: 1.0217x; 1.0217x over previous
import jax
import jax.numpy as jnp
from jax.experimental import pallas as pl
from jax.experimental.pallas import tpu as pltpu

B, N, C, IN, HID, ATOMS = 2, 4096, 1024, 128, 64, 64
TILE_A = 1024
NTA = N // TILE_A
TILE_B = 256
NTB = N // TILE_B


def _phase_a(imp_ref, x_ref, inc_ref, nmw_ref, nmb_ref, cmw_ref, cmb_ref,
             atoms_ref, qw_ref, qb_ref, kw_ref, kb_ref,
             h0_ref, pc_ref, ent_ref, acc_ref, nmw2_ref, cmw2_ref):
    i = pl.program_id(0)

    @pl.when(i == 0)
    def _():
        nmw2_ref[...] = jnp.zeros((B * HID, B * IN), jnp.float32)
        cmw2_ref[...] = jnp.zeros((B * HID, B * IN), jnp.float32)
        nmw2_ref[0:HID, 0:IN] = nmw_ref[...]
        nmw2_ref[HID:B * HID, IN:B * IN] = nmw_ref[...]
        cmw2_ref[0:HID, 0:IN] = cmw_ref[...]
        cmw2_ref[HID:B * HID, IN:B * IN] = cmw_ref[...]

    gate = jax.nn.sigmoid(imp_ref[0, :])
    x_cols = jnp.concatenate([x_ref[0], x_ref[1]], axis=1) * gate[:, None]

    nmb2 = jnp.concatenate([nmb_ref[0, :], nmb_ref[0, :]])
    h0_cols = jax.lax.dot_general(
        x_cols, nmw2_ref[...], (((1,), (1,)), ((), ())),
        preferred_element_type=jnp.float32) + nmb2
    h0_ref[...] = h0_cols

    xc_cols = jax.lax.dot_general(
        x_cols, cmw2_ref[...], (((1,), (1,)), ((), ())),
        preferred_element_type=jnp.float32)
    contrib = jax.lax.dot_general(
        inc_ref[...], xc_cols, (((0,), (0,)), ((), ())),
        preferred_element_type=jnp.float32)

    @pl.when(i == 0)
    def _():
        acc_ref[...] = contrib

    @pl.when(i > 0)
    def _():
        acc_ref[...] += contrib

    @pl.when(i == NTA - 1)
    def _():
        acc = acc_ref[...]
        h2 = jnp.concatenate([acc[:, :HID], acc[:, HID:]], axis=0) \
            + cmb_ref[0, :]
        q = jnp.dot(h2, qw_ref[...].T, preferred_element_type=jnp.float32) \
            + qb_ref[0, :]
        k = jnp.dot(atoms_ref[...], kw_ref[...].T,
                    preferred_element_type=jnp.float32) + kb_ref[0, :]
        attn = jax.lax.dot_general(
            q, k, (((1,), (1,)), ((), ())),
            preferred_element_type=jnp.float32) * (HID ** -0.5)
        m = jnp.max(attn, axis=-1, keepdims=True)
        e = jnp.exp(attn - m)
        s = jnp.sum(e, axis=-1, keepdims=True)
        w = e / s
        pc = jnp.dot(w, atoms_ref[...],
                     preferred_element_type=jnp.float32)
        pc_ref[...] = jnp.concatenate([pc[:C], pc[C:]], axis=1)
        ent = jnp.mean(-jnp.sum(w * jnp.log(w + 1e-6), axis=-1))
        ent_ref[...] = ent.reshape(1, 1)


def _phase_b(adj_ref, h0_ref, inc_ref, pc_ref, sw_ref, sb_ref,
             c1w_ref, c1b_ref, c2w_ref, c2b_ref, pcg_ref, pcb_ref,
             fw_ref, fb_ref, ng_ref, nb_ref, out_ref):
    agg_cols = jnp.dot(adj_ref[...], h0_ref[...],
                       preferred_element_type=jnp.float32)
    pn_cols = jnp.dot(inc_ref[...], pc_ref[...],
                      preferred_element_type=jnp.float32)
    agg = jnp.concatenate([agg_cols[:, :HID], agg_cols[:, HID:]], axis=0)
    pn = jnp.concatenate([pn_cols[:, :HID], pn_cols[:, HID:]], axis=0)
    sur = agg - pn

    err = jnp.sqrt(jnp.sum(sur * sur, axis=-1, keepdims=True))
    conf = 1.0 / (1.0 + err)
    ps = jnp.dot(sur, sw_ref[...].T, preferred_element_type=jnp.float32) \
        + sb_ref[0, :]
    t = jnp.maximum(
        jax.lax.dot_general(jnp.abs(sur), c1w_ref[...],
                            (((1,), (1,)), ((), ())),
                            preferred_element_type=jnp.float32)
        + c1b_ref[0, :], 0.0)
    lc = jax.nn.sigmoid(
        jnp.sum(t * c2w_ref[0:1, :], axis=-1, keepdims=True) + c2b_ref[0, 0])
    gated = ps * (conf * lc)

    h = gated + agg
    mu = jnp.mean(h, axis=-1, keepdims=True)
    va = jnp.mean((h - mu) ** 2, axis=-1, keepdims=True)
    processed = (h - mu) * jax.lax.rsqrt(va + 1e-5) * pcg_ref[0, :] \
        + pcb_ref[0, :]

    fw = fw_ref[...]
    comb = jax.lax.dot_general(processed, fw[:, :HID],
                               (((1,), (1,)), ((), ())),
                               preferred_element_type=jnp.float32) \
        + jax.lax.dot_general(pn, fw[:, HID:],
                              (((1,), (1,)), ((), ())),
                              preferred_element_type=jnp.float32) \
        + fb_ref[0, :]
    mu2 = jnp.mean(comb, axis=-1, keepdims=True)
    va2 = jnp.mean((comb - mu2) ** 2, axis=-1, keepdims=True)
    out = (comb - mu2) * jax.lax.rsqrt(va2 + 1e-5) * ng_ref[0, :] \
        + nb_ref[0, :]
    out_ref[0] = out[0:TILE_B].T
    out_ref[1] = out[TILE_B:B * TILE_B].T


def _full(shape):
    return pl.BlockSpec(shape, lambda i: tuple(0 for _ in shape))


def kernel(x_nodes, adjacency, incidence, node_importance,
           nm_w, nm_b, cm_w, cm_b, atoms, q_w, q_b, k_w, k_b,
           s_w, s_b, c1_w, c1_b, c2_w, c2_b, pc_g, pc_b, f_w, f_b, n_g, n_b):
    imp2 = node_importance.reshape(1, N)
    r = lambda v: v.reshape(1, -1)

    h0_nm, pc_nm, ent = pl.pallas_call(
        _phase_a,
        grid=(NTA,),
        in_specs=[
            pl.BlockSpec((1, TILE_A), lambda i: (0, i)),
            pl.BlockSpec((B, TILE_A, IN), lambda i: (0, i, 0)),
            pl.BlockSpec((TILE_A, C), lambda i: (i, 0)),
            _full((HID, IN)), _full((1, HID)),
            _full((HID, IN)), _full((1, HID)),
            _full((ATOMS, HID)),
            _full((HID, HID)), _full((1, HID)),
            _full((HID, HID)), _full((1, HID)),
        ],
        out_specs=[
            pl.BlockSpec((TILE_A, B * HID), lambda i: (i, 0)),
            _full((C, B * HID)),
            _full((1, 1)),
        ],
        out_shape=[
            jax.ShapeDtypeStruct((N, B * HID), jnp.float32),
            jax.ShapeDtypeStruct((C, B * HID), jnp.float32),
            jax.ShapeDtypeStruct((1, 1), jnp.float32),
        ],
        scratch_shapes=[pltpu.VMEM((C, B * HID), jnp.float32),
                        pltpu.VMEM((B * HID, B * IN), jnp.float32),
                        pltpu.VMEM((B * HID, B * IN), jnp.float32)],
    )(imp2, x_nodes, incidence, nm_w, r(nm_b), cm_w, r(cm_b),
      atoms, q_w, r(q_b), k_w, r(k_b))

    out = pl.pallas_call(
        _phase_b,
        grid=(NTB,),
        in_specs=[
            pl.BlockSpec((TILE_B, N), lambda i: (i, 0)),
            _full((N, B * HID)),
            pl.BlockSpec((TILE_B, C), lambda i: (i, 0)),
            _full((C, B * HID)),
            _full((HID, HID)), _full((1, HID)),
            _full((HID // 4, HID)), _full((1, HID // 4)),
            _full((1, HID // 4)), _full((1, 1)),
            _full((1, HID)), _full((1, HID)),
            _full((HID, B * HID)), _full((1, HID)),
            _full((1, HID)), _full((1, HID)),
        ],
        out_specs=pl.BlockSpec((B, HID, TILE_B), lambda i: (0, 0, i)),
        out_shape=jax.ShapeDtypeStruct((B, HID, N), jnp.float32),
    )(adjacency, h0_nm, incidence, pc_nm, s_w, r(s_b),
      c1_w, r(c1_b), c2_w, c2_b.reshape(1, 1), r(pc_g), r(pc_b),
      f_w, r(f_b), r(n_g), r(n_b))

    return jnp.transpose(out, (0, 2, 1)), ent.reshape(())


# speedup vs baseline: 1.0295x; 1.0076x over previous
import jax
import jax.numpy as jnp
from jax.experimental import pallas as pl
from jax.experimental.pallas import tpu as pltpu

B, N, C, IN, HID, ATOMS = 2, 4096, 1024, 128, 64, 64
TILE_A = 512
NTA = N // TILE_A
TILE_B = 512
NTB = N // TILE_B


def _phase_a(imp_ref, x_ref, inc_ref, nmw_ref, nmb_ref, cmw_ref, cmb_ref,
             atoms_ref, qw_ref, qb_ref, kw_ref, kb_ref,
             h0_ref, pc_ref, ent_ref, acc_ref, nmw2_ref, cmw2_ref):
    i = pl.program_id(0)

    @pl.when(i == 0)
    def _():
        nmw2_ref[...] = jnp.zeros((B * HID, B * IN), jnp.float32)
        cmw2_ref[...] = jnp.zeros((B * HID, B * IN), jnp.float32)
        nmw2_ref[0:HID, 0:IN] = nmw_ref[...]
        nmw2_ref[HID:B * HID, IN:B * IN] = nmw_ref[...]
        cmw2_ref[0:HID, 0:IN] = cmw_ref[...]
        cmw2_ref[HID:B * HID, IN:B * IN] = cmw_ref[...]

    gate = jax.nn.sigmoid(imp_ref[0, :])
    x_cols = jnp.concatenate([x_ref[0], x_ref[1]], axis=1) * gate[:, None]

    nmb2 = jnp.concatenate([nmb_ref[0, :], nmb_ref[0, :]])
    h0_cols = jax.lax.dot_general(
        x_cols, nmw2_ref[...], (((1,), (1,)), ((), ())),
        preferred_element_type=jnp.float32) + nmb2
    h0_ref[...] = h0_cols

    xc_cols = jax.lax.dot_general(
        x_cols, cmw2_ref[...], (((1,), (1,)), ((), ())),
        preferred_element_type=jnp.float32)
    contrib = jax.lax.dot_general(
        inc_ref[...], xc_cols, (((0,), (0,)), ((), ())),
        preferred_element_type=jnp.float32)

    @pl.when(i == 0)
    def _():
        acc_ref[...] = contrib

    @pl.when(i > 0)
    def _():
        acc_ref[...] += contrib

    @pl.when(i == NTA - 1)
    def _():
        acc = acc_ref[...]
        h2 = jnp.concatenate([acc[:, :HID], acc[:, HID:]], axis=0) \
            + cmb_ref[0, :]
        q = jnp.dot(h2, qw_ref[...].T, preferred_element_type=jnp.float32) \
            + qb_ref[0, :]
        k = jnp.dot(atoms_ref[...], kw_ref[...].T,
                    preferred_element_type=jnp.float32) + kb_ref[0, :]
        attn = jax.lax.dot_general(
            q, k, (((1,), (1,)), ((), ())),
            preferred_element_type=jnp.float32) * (HID ** -0.5)
        m = jnp.max(attn, axis=-1, keepdims=True)
        e = jnp.exp(attn - m)
        s = jnp.sum(e, axis=-1, keepdims=True)
        w = e / s
        pc = jnp.dot(w, atoms_ref[...],
                     preferred_element_type=jnp.float32)
        pc_ref[...] = jnp.concatenate([pc[:C], pc[C:]], axis=1)
        ent = jnp.mean(-jnp.sum(w * jnp.log(w + 1e-6), axis=-1))
        ent_ref[...] = ent.reshape(1, 1)


def _phase_b(adj_ref, h0_ref, inc_ref, pc_ref, sw_ref, sb_ref,
             c1w_ref, c1b_ref, c2w_ref, c2b_ref, pcg_ref, pcb_ref,
             fw_ref, fb_ref, ng_ref, nb_ref, out_ref):
    agg_cols = jnp.dot(adj_ref[...], h0_ref[...],
                       preferred_element_type=jnp.float32)
    pn_cols = jnp.dot(inc_ref[...], pc_ref[...],
                      preferred_element_type=jnp.float32)
    agg = jnp.concatenate([agg_cols[:, :HID], agg_cols[:, HID:]], axis=0)
    pn = jnp.concatenate([pn_cols[:, :HID], pn_cols[:, HID:]], axis=0)
    sur = agg - pn

    err = jnp.sqrt(jnp.sum(sur * sur, axis=-1, keepdims=True))
    conf = 1.0 / (1.0 + err)
    ps = jnp.dot(sur, sw_ref[...].T, preferred_element_type=jnp.float32) \
        + sb_ref[0, :]
    t = jnp.maximum(
        jax.lax.dot_general(jnp.abs(sur), c1w_ref[...],
                            (((1,), (1,)), ((), ())),
                            preferred_element_type=jnp.float32)
        + c1b_ref[0, :], 0.0)
    lc = jax.nn.sigmoid(
        jnp.sum(t * c2w_ref[0:1, :], axis=-1, keepdims=True) + c2b_ref[0, 0])
    gated = ps * (conf * lc)

    h = gated + agg
    mu = jnp.mean(h, axis=-1, keepdims=True)
    va = jnp.mean((h - mu) ** 2, axis=-1, keepdims=True)
    processed = (h - mu) * jax.lax.rsqrt(va + 1e-5) * pcg_ref[0, :] \
        + pcb_ref[0, :]

    fw = fw_ref[...]
    comb = jax.lax.dot_general(processed, fw[:, :HID],
                               (((1,), (1,)), ((), ())),
                               preferred_element_type=jnp.float32) \
        + jax.lax.dot_general(pn, fw[:, HID:],
                              (((1,), (1,)), ((), ())),
                              preferred_element_type=jnp.float32) \
        + fb_ref[0, :]
    mu2 = jnp.mean(comb, axis=-1, keepdims=True)
    va2 = jnp.mean((comb - mu2) ** 2, axis=-1, keepdims=True)
    out = (comb - mu2) * jax.lax.rsqrt(va2 + 1e-5) * ng_ref[0, :] \
        + nb_ref[0, :]
    out_ref[0] = out[0:TILE_B].T
    out_ref[1] = out[TILE_B:B * TILE_B].T


def _full(shape):
    return pl.BlockSpec(shape, lambda i: tuple(0 for _ in shape))


def kernel(x_nodes, adjacency, incidence, node_importance,
           nm_w, nm_b, cm_w, cm_b, atoms, q_w, q_b, k_w, k_b,
           s_w, s_b, c1_w, c1_b, c2_w, c2_b, pc_g, pc_b, f_w, f_b, n_g, n_b):
    imp2 = node_importance.reshape(1, N)
    r = lambda v: v.reshape(1, -1)

    h0_nm, pc_nm, ent = pl.pallas_call(
        _phase_a,
        grid=(NTA,),
        in_specs=[
            pl.BlockSpec((1, TILE_A), lambda i: (0, i)),
            pl.BlockSpec((B, TILE_A, IN), lambda i: (0, i, 0)),
            pl.BlockSpec((TILE_A, C), lambda i: (i, 0)),
            _full((HID, IN)), _full((1, HID)),
            _full((HID, IN)), _full((1, HID)),
            _full((ATOMS, HID)),
            _full((HID, HID)), _full((1, HID)),
            _full((HID, HID)), _full((1, HID)),
        ],
        out_specs=[
            pl.BlockSpec((TILE_A, B * HID), lambda i: (i, 0)),
            _full((C, B * HID)),
            _full((1, 1)),
        ],
        out_shape=[
            jax.ShapeDtypeStruct((N, B * HID), jnp.float32),
            jax.ShapeDtypeStruct((C, B * HID), jnp.float32),
            jax.ShapeDtypeStruct((1, 1), jnp.float32),
        ],
        scratch_shapes=[pltpu.VMEM((C, B * HID), jnp.float32),
                        pltpu.VMEM((B * HID, B * IN), jnp.float32),
                        pltpu.VMEM((B * HID, B * IN), jnp.float32)],
    )(imp2, x_nodes, incidence, nm_w, r(nm_b), cm_w, r(cm_b),
      atoms, q_w, r(q_b), k_w, r(k_b))

    out = pl.pallas_call(
        _phase_b,
        grid=(NTB,),
        in_specs=[
            pl.BlockSpec((TILE_B, N), lambda i: (i, 0)),
            _full((N, B * HID)),
            pl.BlockSpec((TILE_B, C), lambda i: (i, 0)),
            _full((C, B * HID)),
            _full((HID, HID)), _full((1, HID)),
            _full((HID // 4, HID)), _full((1, HID // 4)),
            _full((1, HID // 4)), _full((1, 1)),
            _full((1, HID)), _full((1, HID)),
            _full((HID, B * HID)), _full((1, HID)),
            _full((1, HID)), _full((1, HID)),
        ],
        out_specs=pl.BlockSpec((B, HID, TILE_B), lambda i: (0, 0, i)),
        out_shape=jax.ShapeDtypeStruct((B, HID, N), jnp.float32),
    )(adjacency, h0_nm, incidence, pc_nm, s_w, r(s_b),
      c1_w, r(c1_b), c2_w, c2_b.reshape(1, 1), r(pc_g), r(pc_b),
      f_w, r(f_b), r(n_g), r(n_b))

    return jnp.transpose(out, (0, 2, 1)), ent.reshape(())


# speedup vs baseline: 1.1626x; 1.1293x over previous
import jax
import jax.numpy as jnp
from jax.experimental import pallas as pl
from jax.experimental.pallas import tpu as pltpu

B, N, C, IN, HID, ATOMS = 2, 4096, 1024, 128, 64, 64
TILE_A = 1024
NTA = N // TILE_A
TILE_B = 512
NTB = N // TILE_B


def _fused(imp_ref, x_ref, incA_ref, adj_ref, incB_ref,
           nmw_ref, nmb_ref, cmw_ref, cmb_ref,
           atoms_ref, qw_ref, qb_ref, kw_ref, kb_ref,
           sw_ref, sb_ref, c1w_ref, c1b_ref, c2w_ref, c2b_ref,
           pcg_ref, pcb_ref, fw_ref, fb_ref, ng_ref, nb_ref,
           out_ref, ent_ref,
           h0_s, acc_s, pc_s, nmw2_s, cmw2_s):
    s = pl.program_id(0)

    @pl.when(s == 0)
    def _():
        nmw2_s[...] = jnp.zeros((B * HID, B * IN), jnp.float32)
        cmw2_s[...] = jnp.zeros((B * HID, B * IN), jnp.float32)
        nmw2_s[0:HID, 0:IN] = nmw_ref[...]
        nmw2_s[HID:B * HID, IN:B * IN] = nmw_ref[...]
        cmw2_s[0:HID, 0:IN] = cmw_ref[...]
        cmw2_s[HID:B * HID, IN:B * IN] = cmw_ref[...]

    @pl.when(s < NTA)
    def _phase_a():
        gate = jax.nn.sigmoid(imp_ref[0, :])
        x_cols = jnp.concatenate([x_ref[0], x_ref[1]], axis=1) * gate[:, None]

        nmb2 = jnp.concatenate([nmb_ref[0, :], nmb_ref[0, :]])
        h0_cols = jax.lax.dot_general(
            x_cols, nmw2_s[...], (((1,), (1,)), ((), ())),
            preferred_element_type=jnp.float32) + nmb2
        h0_s[pl.ds(s * TILE_A, TILE_A), :] = h0_cols

        xc_cols = jax.lax.dot_general(
            x_cols, cmw2_s[...], (((1,), (1,)), ((), ())),
            preferred_element_type=jnp.float32)
        contrib = jax.lax.dot_general(
            incA_ref[...], xc_cols, (((0,), (0,)), ((), ())),
            preferred_element_type=jnp.float32)

        @pl.when(s == 0)
        def _():
            acc_s[...] = contrib

        @pl.when(s > 0)
        def _():
            acc_s[...] += contrib

        @pl.when(s == NTA - 1)
        def _attention():
            acc = acc_s[...]
            h2 = jnp.concatenate([acc[:, :HID], acc[:, HID:]], axis=0) \
                + cmb_ref[0, :]
            q = jnp.dot(h2, qw_ref[...].T,
                        preferred_element_type=jnp.float32) + qb_ref[0, :]
            k = jnp.dot(atoms_ref[...], kw_ref[...].T,
                        preferred_element_type=jnp.float32) + kb_ref[0, :]
            attn = jax.lax.dot_general(
                q, k, (((1,), (1,)), ((), ())),
                preferred_element_type=jnp.float32) * (HID ** -0.5)
            m = jnp.max(attn, axis=-1, keepdims=True)
            e = jnp.exp(attn - m)
            t = jnp.sum(e, axis=-1, keepdims=True)
            w = e / t
            pc = jnp.dot(w, atoms_ref[...],
                         preferred_element_type=jnp.float32)
            pc_s[...] = jnp.concatenate([pc[:C], pc[C:]], axis=1)
            ent = jnp.mean(-jnp.sum(w * jnp.log(w + 1e-6), axis=-1))
            ent_ref[...] = ent.reshape(1, 1)

    @pl.when(s >= NTA)
    def _phase_b():
        agg_cols = jnp.dot(adj_ref[...], h0_s[...],
                           preferred_element_type=jnp.float32)
        pn_cols = jnp.dot(incB_ref[...], pc_s[...],
                          preferred_element_type=jnp.float32)
        agg = jnp.concatenate([agg_cols[:, :HID], agg_cols[:, HID:]], axis=0)
        pn = jnp.concatenate([pn_cols[:, :HID], pn_cols[:, HID:]], axis=0)
        sur = agg - pn

        err = jnp.sqrt(jnp.sum(sur * sur, axis=-1, keepdims=True))
        conf = 1.0 / (1.0 + err)
        ps = jnp.dot(sur, sw_ref[...].T,
                     preferred_element_type=jnp.float32) + sb_ref[0, :]
        t = jnp.maximum(
            jax.lax.dot_general(jnp.abs(sur), c1w_ref[...],
                                (((1,), (1,)), ((), ())),
                                preferred_element_type=jnp.float32)
            + c1b_ref[0, :], 0.0)
        lc = jax.nn.sigmoid(
            jnp.sum(t * c2w_ref[0:1, :], axis=-1, keepdims=True)
            + c2b_ref[0, 0])
        gated = ps * (conf * lc)

        h = gated + agg
        mu = jnp.mean(h, axis=-1, keepdims=True)
        va = jnp.mean((h - mu) ** 2, axis=-1, keepdims=True)
        processed = (h - mu) * jax.lax.rsqrt(va + 1e-5) * pcg_ref[0, :] \
            + pcb_ref[0, :]

        fw = fw_ref[...]
        comb = jax.lax.dot_general(processed, fw[:, :HID],
                                   (((1,), (1,)), ((), ())),
                                   preferred_element_type=jnp.float32) \
            + jax.lax.dot_general(pn, fw[:, HID:],
                                  (((1,), (1,)), ((), ())),
                                  preferred_element_type=jnp.float32) \
            + fb_ref[0, :]
        mu2 = jnp.mean(comb, axis=-1, keepdims=True)
        va2 = jnp.mean((comb - mu2) ** 2, axis=-1, keepdims=True)
        out = (comb - mu2) * jax.lax.rsqrt(va2 + 1e-5) * ng_ref[0, :] \
            + nb_ref[0, :]
        out_ref[0] = out[0:TILE_B].T
        out_ref[1] = out[TILE_B:B * TILE_B].T


def _full(shape):
    return pl.BlockSpec(shape, lambda s: tuple(0 for _ in shape))


def kernel(x_nodes, adjacency, incidence, node_importance,
           nm_w, nm_b, cm_w, cm_b, atoms, q_w, q_b, k_w, k_b,
           s_w, s_b, c1_w, c1_b, c2_w, c2_b, pc_g, pc_b, f_w, f_b, n_g, n_b):
    imp2 = node_importance.reshape(1, N)
    r = lambda v: v.reshape(1, -1)

    a_idx = lambda s: jnp.minimum(s, NTA - 1)
    b_idx = lambda s: jnp.maximum(s - NTA, 0)

    out, ent = pl.pallas_call(
        _fused,
        grid=(NTA + NTB,),
        in_specs=[
            pl.BlockSpec((1, TILE_A), lambda s: (0, a_idx(s))),
            pl.BlockSpec((B, TILE_A, IN), lambda s: (0, a_idx(s), 0)),
            pl.BlockSpec((TILE_A, C), lambda s: (a_idx(s), 0)),
            pl.BlockSpec((TILE_B, N), lambda s: (b_idx(s), 0)),
            pl.BlockSpec((TILE_B, C), lambda s: (b_idx(s), 0)),
            _full((HID, IN)), _full((1, HID)),
            _full((HID, IN)), _full((1, HID)),
            _full((ATOMS, HID)),
            _full((HID, HID)), _full((1, HID)),
            _full((HID, HID)), _full((1, HID)),
            _full((HID, HID)), _full((1, HID)),
            _full((HID // 4, HID)), _full((1, HID // 4)),
            _full((1, HID // 4)), _full((1, 1)),
            _full((1, HID)), _full((1, HID)),
            _full((HID, B * HID)), _full((1, HID)),
            _full((1, HID)), _full((1, HID)),
        ],
        out_specs=[
            pl.BlockSpec((B, HID, TILE_B), lambda s: (0, 0, b_idx(s))),
            _full((1, 1)),
        ],
        out_shape=[
            jax.ShapeDtypeStruct((B, HID, N), jnp.float32),
            jax.ShapeDtypeStruct((1, 1), jnp.float32),
        ],
        scratch_shapes=[
            pltpu.VMEM((N, B * HID), jnp.float32),
            pltpu.VMEM((C, B * HID), jnp.float32),
            pltpu.VMEM((C, B * HID), jnp.float32),
            pltpu.VMEM((B * HID, B * IN), jnp.float32),
            pltpu.VMEM((B * HID, B * IN), jnp.float32),
        ],
    )(imp2, x_nodes, incidence, adjacency, incidence,
      nm_w, r(nm_b), cm_w, r(cm_b),
      atoms, q_w, r(q_b), k_w, r(k_b),
      s_w, r(s_b), c1_w, r(c1_b), c2_w, c2_b.reshape(1, 1),
      r(pc_g), r(pc_b), f_w, r(f_b), r(n_g), r(n_b))

    return jnp.transpose(out, (0, 2, 1)), ent.reshape(())
